# Initial kernel scaffold; baseline (speedup 1.0000x reference)
import jax
import jax.numpy as jnp
from jax.experimental import pallas as pl


def kernel(x, coefficients_vect):
    raise NotImplementedError("write your pallas kernel here")


# speedup vs baseline: 900.9140x; 900.9140x over previous
import functools

import jax
import jax.numpy as jnp
from jax import lax
from jax.experimental import pallas as pl
from jax.experimental.pallas import tpu as pltpu
from jax.experimental.pallas import tpu_sc as plsc

_N_CHANNELS = 192
_N_KNOTS = 63
_T_RANGE = 4.0


def _grid_value():
    round_to = 1e-06
    return float(_T_RANGE) / (_N_KNOTS // 2) // round_to * round_to


def kernel(x, coefficients_vect):
    B, C, H, W = x.shape
    n_rows = B * C
    row_len = H * W
    n_workers = 32
    rows_per_w = n_rows // n_workers

    grid = _grid_value()
    inv_grid = 1.0 / grid
    lo = float(-(_N_KNOTS // 2))
    hi = float(_N_KNOTS // 2 - 2)
    half = _N_KNOTS // 2

    x2 = x.reshape(n_rows, row_len)

    mesh = plsc.VectorSubcoreMesh(core_axis_name="c", subcore_axis_name="s")

    @functools.partial(
        pl.kernel,
        mesh=mesh,
        out_type=jax.ShapeDtypeStruct((n_rows, row_len), jnp.float32),
        scratch_types=[
            pltpu.VMEM((_N_CHANNELS * _N_KNOTS,), jnp.float32),
            pltpu.VMEM((row_len,), jnp.float32),
            pltpu.VMEM((row_len,), jnp.float32),
        ],
        compiler_params=pltpu.CompilerParams(needs_layout_passes=False),
    )
    def spline_sc(x_hbm, coef_hbm, out_hbm, table, xbuf, obuf):
        wid = lax.axis_index("s") * 2 + lax.axis_index("c")
        pltpu.sync_copy(coef_hbm, table)
        row0 = wid * rows_per_w

        def row_body(i, carry):
            row = row0 + i
            c = lax.rem(row, _N_CHANNELS)
            base = c * _N_KNOTS + half
            pltpu.sync_copy(x_hbm.at[row], xbuf)

            def vec_body(j, carry2):
                xv = xbuf[pl.ds(j * 16, 16)]
                ti = (xv * inv_grid + 64.0).astype(jnp.int32)
                ti = jnp.minimum(jnp.maximum(ti, 64 - half), 64 + half - 2)
                fl = ti.astype(jnp.float32) - 64.0
                idx = ti + (base - 64)
                s = (xv - fl * grid) * inv_grid
                s2 = s * s
                frac3 = s2 * 0.5
                frac1 = frac3 - s + 0.5
                frac2 = (s - s2) + 0.5
                g0 = plsc.load_gather(table, [idx])
                g1 = plsc.load_gather(table, [idx + 1])
                g2 = plsc.load_gather(table, [idx + 2])
                obuf[pl.ds(j * 16, 16)] = g2 * frac3 + g1 * frac2 + g0 * frac1
                return carry2

            lax.fori_loop(0, row_len // 16, vec_body, 0)
            pltpu.sync_copy(obuf, out_hbm.at[row])
            return carry

        lax.fori_loop(0, rows_per_w, row_body, 0)

    out = spline_sc(x2, coefficients_vect)
    return out.reshape(B, C, H, W)


# speedup vs baseline: 1133.4966x; 1.2582x over previous
import functools

import jax
import jax.numpy as jnp
from jax import lax
from jax.experimental import pallas as pl
from jax.experimental.pallas import tpu as pltpu
from jax.experimental.pallas import tpu_sc as plsc

_N_CHANNELS = 192
_N_KNOTS = 63
_T_RANGE = 4.0


def _grid_value():
    round_to = 1e-06
    return float(_T_RANGE) / (_N_KNOTS // 2) // round_to * round_to


def kernel(x, coefficients_vect):
    B, C, H, W = x.shape
    n_rows = B * C
    row_len = H * W
    n_workers = 32
    rows_per_w = n_rows // n_workers

    grid = _grid_value()
    inv_grid = 1.0 / grid
    lo = float(-(_N_KNOTS // 2))
    hi = float(_N_KNOTS // 2 - 2)
    half = _N_KNOTS // 2

    x2 = x.reshape(n_rows, row_len)

    mesh = plsc.VectorSubcoreMesh(core_axis_name="c", subcore_axis_name="s")

    @functools.partial(
        pl.kernel,
        mesh=mesh,
        out_type=jax.ShapeDtypeStruct((n_rows, row_len), jnp.float32),
        scratch_types=[
            pltpu.VMEM((_N_CHANNELS * _N_KNOTS,), jnp.float32),
            pltpu.VMEM((row_len,), jnp.float32),
            pltpu.VMEM((row_len,), jnp.float32),
        ],
        compiler_params=pltpu.CompilerParams(needs_layout_passes=False),
    )
    def spline_sc(x_hbm, coef_hbm, out_hbm, table, xbuf, obuf):
        wid = lax.axis_index("s") * 2 + lax.axis_index("c")
        pltpu.sync_copy(coef_hbm, table)
        row0 = wid * rows_per_w

        def row_body(i, carry):
            row = row0 + i
            c = lax.rem(row, _N_CHANNELS)
            base = c * _N_KNOTS + half
            pltpu.sync_copy(x_hbm.at[row], xbuf)

            @plsc.parallel_loop(0, row_len, 16, unroll=8)
            def vec_body(j):
                xv = xbuf[pl.ds(j, 16)]
                ti = (xv * inv_grid + 64.0).astype(jnp.int32)
                ti = jnp.minimum(jnp.maximum(ti, 64 - half), 64 + half - 2)
                fl = ti.astype(jnp.float32) - 64.0
                idx = ti + (base - 64)
                s = (xv - fl * grid) * inv_grid
                s2 = s * s
                frac3 = s2 * 0.5
                frac1 = frac3 - s + 0.5
                frac2 = (s - s2) + 0.5
                g0 = plsc.load_gather(table, [idx])
                g1 = plsc.load_gather(table, [idx + 1])
                g2 = plsc.load_gather(table, [idx + 2])
                obuf[pl.ds(j, 16)] = g2 * frac3 + g1 * frac2 + g0 * frac1

            pltpu.sync_copy(obuf, out_hbm.at[row])
            return carry

        lax.fori_loop(0, rows_per_w, row_body, 0)

    out = spline_sc(x2, coefficients_vect)
    return out.reshape(B, C, H, W)


# speedup vs baseline: 1408.3873x; 1.2425x over previous
import functools

import jax
import jax.numpy as jnp
from jax import lax
from jax.experimental import pallas as pl
from jax.experimental.pallas import tpu as pltpu
from jax.experimental.pallas import tpu_sc as plsc

_N_CHANNELS = 192
_N_KNOTS = 63
_T_RANGE = 4.0


def _grid_value():
    round_to = 1e-06
    return float(_T_RANGE) / (_N_KNOTS // 2) // round_to * round_to


def kernel(x, coefficients_vect):
    B, C, H, W = x.shape
    n_rows = B * C
    row_len = H * W
    n_workers = 32
    rows_per_w = n_rows // n_workers

    grid = _grid_value()
    inv_grid = 1.0 / grid
    half = _N_KNOTS // 2
    n_tab = _N_CHANNELS * _N_KNOTS

    g0 = coefficients_vect
    g1 = jnp.concatenate([coefficients_vect[1:], jnp.zeros((1,), jnp.float32)])
    g2 = jnp.concatenate([coefficients_vect[2:], jnp.zeros((2,), jnp.float32)])
    fl = (jnp.arange(n_tab, dtype=jnp.float32) % _N_KNOTS) - float(half)
    a = 0.5 * (g0 + g1)
    b = g1 - g0
    c2 = 0.5 * (g0 + g2) - g1
    p2_t = c2
    p1_t = b - 2.0 * c2 * fl
    p0_t = a - b * fl + c2 * fl * fl

    x2 = x.reshape(n_rows, row_len)

    mesh = plsc.VectorSubcoreMesh(core_axis_name="c", subcore_axis_name="s")

    @functools.partial(
        pl.kernel,
        mesh=mesh,
        out_type=jax.ShapeDtypeStruct((n_rows, row_len), jnp.float32),
        scratch_types=[
            pltpu.VMEM((n_tab,), jnp.float32),
            pltpu.VMEM((n_tab,), jnp.float32),
            pltpu.VMEM((n_tab,), jnp.float32),
            pltpu.VMEM((row_len,), jnp.float32),
            pltpu.VMEM((row_len,), jnp.float32),
        ],
        compiler_params=pltpu.CompilerParams(needs_layout_passes=False),
    )
    def spline_sc(x_hbm, p0_hbm, p1_hbm, p2_hbm, out_hbm,
                  tab0, tab1, tab2, xbuf, obuf):
        wid = lax.axis_index("s") * 2 + lax.axis_index("c")
        pltpu.sync_copy(p0_hbm, tab0)
        pltpu.sync_copy(p1_hbm, tab1)
        pltpu.sync_copy(p2_hbm, tab2)
        row0 = wid * rows_per_w

        def row_body(i, carry):
            row = row0 + i
            c = lax.rem(row, _N_CHANNELS)
            base = c * _N_KNOTS + half - 64
            pltpu.sync_copy(x_hbm.at[row], xbuf)

            @plsc.parallel_loop(0, row_len, 16, unroll=8)
            def vec_body(j):
                xv = xbuf[pl.ds(j, 16)]
                q = xv * inv_grid
                ti = (q + 64.0).astype(jnp.int32)
                ti = jnp.minimum(jnp.maximum(ti, 64 - half), 64 + half - 2)
                idx = ti + base
                p0 = plsc.load_gather(tab0, [idx])
                p1 = plsc.load_gather(tab1, [idx])
                p2 = plsc.load_gather(tab2, [idx])
                obuf[pl.ds(j, 16)] = p0 + q * (p1 + q * p2)

            pltpu.sync_copy(obuf, out_hbm.at[row])
            return carry

        lax.fori_loop(0, rows_per_w, row_body, 0)

    out = spline_sc(x2, p0_t, p1_t, p2_t)
    return out.reshape(B, C, H, W)


# speedup vs baseline: 1876.7706x; 1.3326x over previous
import functools

import jax
import jax.numpy as jnp
from jax import lax
from jax.experimental import pallas as pl
from jax.experimental.pallas import tpu as pltpu
from jax.experimental.pallas import tpu_sc as plsc

_N_CHANNELS = 192
_N_KNOTS = 63
_T_RANGE = 4.0


def _grid_value():
    round_to = 1e-06
    return float(_T_RANGE) / (_N_KNOTS // 2) // round_to * round_to


def kernel(x, coefficients_vect):
    B, C, H, W = x.shape
    n_rows = B * C
    row_len = H * W
    n_workers = 32
    rows_per_w = n_rows // n_workers

    grid = _grid_value()
    inv_grid = 1.0 / grid
    half = _N_KNOTS // 2
    n_tab = _N_CHANNELS * _N_KNOTS

    g0 = coefficients_vect
    g1 = jnp.concatenate([coefficients_vect[1:], jnp.zeros((1,), jnp.float32)])
    g2 = jnp.concatenate([coefficients_vect[2:], jnp.zeros((2,), jnp.float32)])
    fl = (jnp.arange(n_tab, dtype=jnp.float32) % _N_KNOTS) - float(half)
    a = 0.5 * (g0 + g1)
    b = g1 - g0
    c2 = 0.5 * (g0 + g2) - g1
    p2_t = c2
    p1_t = b - 2.0 * c2 * fl
    p0_t = a - b * fl + c2 * fl * fl

    x2 = x.reshape(n_rows, row_len)

    mesh = plsc.VectorSubcoreMesh(core_axis_name="c", subcore_axis_name="s")

    @functools.partial(
        pl.kernel,
        mesh=mesh,
        out_type=jax.ShapeDtypeStruct((n_rows, row_len), jnp.float32),
        scratch_types=[
            pltpu.VMEM((n_tab,), jnp.float32),
            pltpu.VMEM((n_tab,), jnp.float32),
            pltpu.VMEM((n_tab,), jnp.float32),
            pltpu.VMEM((row_len,), jnp.float32),
            pltpu.VMEM((row_len,), jnp.float32),
            pltpu.VMEM((row_len,), jnp.float32),
            pltpu.VMEM((row_len,), jnp.float32),
            pltpu.SemaphoreType.DMA,
            pltpu.SemaphoreType.DMA,
            pltpu.SemaphoreType.DMA,
            pltpu.SemaphoreType.DMA,
        ],
        compiler_params=pltpu.CompilerParams(needs_layout_passes=False),
    )
    def spline_sc(x_hbm, p0_hbm, p1_hbm, p2_hbm, out_hbm,
                  tab0, tab1, tab2, xb0, xb1, ob0, ob1,
                  sin0, sin1, sout0, sout1):
        wid = lax.axis_index("s") * 2 + lax.axis_index("c")
        pltpu.sync_copy(p0_hbm, tab0)
        pltpu.sync_copy(p1_hbm, tab1)
        pltpu.sync_copy(p2_hbm, tab2)
        row0 = wid * rows_per_w
        xbufs, obufs = (xb0, xb1), (ob0, ob1)
        sins, souts = (sin0, sin1), (sout0, sout1)
        n_pairs = rows_per_w // 2

        pltpu.async_copy(x_hbm.at[row0], xb0, sin0)
        pltpu.async_copy(x_hbm.at[row0 + 1], xb1, sin1)

        def pair_body(g, carry):
            for bi in range(2):
                row = row0 + g * 2 + bi
                xbuf, obuf = xbufs[bi], obufs[bi]
                sin, sout = sins[bi], souts[bi]
                pltpu.make_async_copy(x_hbm.at[row], xbuf, sin).wait()

                @pl.when(g > 0)
                def _wait_out():
                    pltpu.make_async_copy(obuf, out_hbm.at[row], sout).wait()

                c = lax.rem(row, _N_CHANNELS)
                base = c * _N_KNOTS + half - 64

                @plsc.parallel_loop(0, row_len, 16, unroll=8)
                def vec_body(j):
                    xv = xbuf[pl.ds(j, 16)]
                    q = xv * inv_grid
                    ti = (q + 64.0).astype(jnp.int32)
                    ti = jnp.minimum(jnp.maximum(ti, 64 - half), 64 + half - 2)
                    idx = ti + base
                    p0 = plsc.load_gather(tab0, [idx])
                    p1 = plsc.load_gather(tab1, [idx])
                    p2 = plsc.load_gather(tab2, [idx])
                    obuf[pl.ds(j, 16)] = p0 + q * (p1 + q * p2)

                pltpu.async_copy(obuf, out_hbm.at[row], sout)

                @pl.when(g < n_pairs - 1)
                def _next_in():
                    pltpu.async_copy(x_hbm.at[row + 2], xbuf, sin)
            return carry

        lax.fori_loop(0, n_pairs, pair_body, 0)
        last = row0 + rows_per_w - 2
        pltpu.make_async_copy(ob0, out_hbm.at[last], sout0).wait()
        pltpu.make_async_copy(ob1, out_hbm.at[last + 1], sout1).wait()

    out = spline_sc(x2, p0_t, p1_t, p2_t)
    return out.reshape(B, C, H, W)


# speedup vs baseline: 4292.0093x; 2.2869x over previous
import functools

import jax
import jax.numpy as jnp
from jax import lax
from jax.experimental import pallas as pl
from jax.experimental.pallas import tpu as pltpu
from jax.experimental.pallas import tpu_sc as plsc

_N_CHANNELS = 192
_N_KNOTS = 63
_T_RANGE = 4.0


def _grid_value():
    round_to = 1e-06
    return float(_T_RANGE) / (_N_KNOTS // 2) // round_to * round_to


def kernel(x, coefficients_vect):
    B, C, H, W = x.shape
    n_pages = B * C
    n_workers = 32
    pages_per_w = n_pages // n_workers

    grid = _grid_value()
    inv_grid = 1.0 / grid
    half = _N_KNOTS // 2
    n_tab = _N_CHANNELS * _N_KNOTS

    g0 = coefficients_vect
    g1 = jnp.concatenate([coefficients_vect[1:], jnp.zeros((1,), jnp.float32)])
    g2 = jnp.concatenate([coefficients_vect[2:], jnp.zeros((2,), jnp.float32)])
    fl = (jnp.arange(n_tab, dtype=jnp.float32) % _N_KNOTS) - float(half)
    a = 0.5 * (g0 + g1)
    b = g1 - g0
    c2 = 0.5 * (g0 + g2) - g1
    p2_t = c2
    p1_t = b - 2.0 * c2 * fl
    p0_t = a - b * fl + c2 * fl * fl

    mesh = plsc.VectorSubcoreMesh(core_axis_name="c", subcore_axis_name="s")

    @functools.partial(
        pl.kernel,
        mesh=mesh,
        out_type=jax.ShapeDtypeStruct((B, C, H, W), jnp.float32),
        scratch_types=[
            pltpu.VMEM((n_tab,), jnp.float32),
            pltpu.VMEM((n_tab,), jnp.float32),
            pltpu.VMEM((n_tab,), jnp.float32),
            pltpu.VMEM((H, W), jnp.float32),
            pltpu.VMEM((H, W), jnp.float32),
            pltpu.VMEM((H, W), jnp.float32),
            pltpu.VMEM((H, W), jnp.float32),
            pltpu.SemaphoreType.DMA,
            pltpu.SemaphoreType.DMA,
            pltpu.SemaphoreType.DMA,
            pltpu.SemaphoreType.DMA,
        ],
        compiler_params=pltpu.CompilerParams(needs_layout_passes=False),
    )
    def spline_sc(x_hbm, p0_hbm, p1_hbm, p2_hbm, out_hbm,
                  tab0, tab1, tab2, xb0, xb1, ob0, ob1,
                  sin0, sin1, sout0, sout1):
        wid = lax.axis_index("s") * 2 + lax.axis_index("c")
        pltpu.sync_copy(p0_hbm, tab0)
        pltpu.sync_copy(p1_hbm, tab1)
        pltpu.sync_copy(p2_hbm, tab2)
        page0 = wid * pages_per_w
        xbufs, obufs = (xb0, xb1), (ob0, ob1)
        sins, souts = (sin0, sin1), (sout0, sout1)
        n_pairs = pages_per_w // 2

        def page_bc(p):
            return lax.div(p, _N_CHANNELS), lax.rem(p, _N_CHANNELS)

        b0, c0 = page_bc(page0)
        pltpu.async_copy(x_hbm.at[b0, c0], xb0, sin0)
        b1, c1 = page_bc(page0 + 1)
        pltpu.async_copy(x_hbm.at[b1, c1], xb1, sin1)

        def pair_body(g, carry):
            for bi in range(2):
                page = page0 + g * 2 + bi
                pb, pc = page_bc(page)
                xbuf, obuf = xbufs[bi], obufs[bi]
                sin, sout = sins[bi], souts[bi]
                pltpu.make_async_copy(x_hbm.at[pb, pc], xbuf, sin).wait()

                @pl.when(g > 0)
                def _wait_out():
                    pltpu.make_async_copy(obuf, out_hbm.at[pb, pc], sout).wait()

                base = pc * _N_KNOTS + half - 64

                @plsc.parallel_loop(0, H, 1, unroll=2)
                def vec_body(r):
                    for cc in range(W // 16):
                        xv = xbuf[r, pl.ds(cc * 16, 16)]
                        q = xv * inv_grid
                        ti = (q + 64.0).astype(jnp.int32)
                        ti = jnp.minimum(jnp.maximum(ti, 64 - half),
                                         64 + half - 2)
                        idx = ti + base
                        p0 = plsc.load_gather(tab0, [idx])
                        p1 = plsc.load_gather(tab1, [idx])
                        p2 = plsc.load_gather(tab2, [idx])
                        obuf[r, pl.ds(cc * 16, 16)] = p0 + q * (p1 + q * p2)

                pltpu.async_copy(obuf, out_hbm.at[pb, pc], sout)

                @pl.when(g < n_pairs - 1)
                def _next_in():
                    nb, nc = page_bc(page + 2)
                    pltpu.async_copy(x_hbm.at[nb, nc], xbuf, sin)
            return carry

        lax.fori_loop(0, n_pairs, pair_body, 0)
        lb0, lc0 = page_bc(page0 + pages_per_w - 2)
        pltpu.make_async_copy(ob0, out_hbm.at[lb0, lc0], sout0).wait()
        lb1, lc1 = page_bc(page0 + pages_per_w - 1)
        pltpu.make_async_copy(ob1, out_hbm.at[lb1, lc1], sout1).wait()

    return spline_sc(x, p0_t, p1_t, p2_t)


# speedup vs baseline: 4314.5541x; 1.0053x over previous
import functools

import jax
import jax.numpy as jnp
from jax import lax
from jax.experimental import pallas as pl
from jax.experimental.pallas import tpu as pltpu
from jax.experimental.pallas import tpu_sc as plsc

_N_CHANNELS = 192
_N_KNOTS = 63
_T_RANGE = 4.0


def _grid_value():
    round_to = 1e-06
    return float(_T_RANGE) / (_N_KNOTS // 2) // round_to * round_to


def kernel(x, coefficients_vect):
    B, C, H, W = x.shape
    n_pages = B * C
    n_workers = 32
    pages_per_w = n_pages // n_workers

    grid = _grid_value()
    inv_grid = 1.0 / grid
    half = _N_KNOTS // 2
    n_tab = _N_CHANNELS * _N_KNOTS

    g0 = coefficients_vect
    g1 = jnp.concatenate([coefficients_vect[1:], jnp.zeros((1,), jnp.float32)])
    g2 = jnp.concatenate([coefficients_vect[2:], jnp.zeros((2,), jnp.float32)])
    fl = (jnp.arange(n_tab, dtype=jnp.float32) % _N_KNOTS) - float(half)
    a = 0.5 * (g0 + g1)
    b = g1 - g0
    c2 = 0.5 * (g0 + g2) - g1
    p2_t = c2
    p1_t = b - 2.0 * c2 * fl
    p0_t = a - b * fl + c2 * fl * fl

    mesh = plsc.VectorSubcoreMesh(core_axis_name="c", subcore_axis_name="s")

    @functools.partial(
        pl.kernel,
        mesh=mesh,
        out_type=jax.ShapeDtypeStruct((B, C, H, W), jnp.float32),
        scratch_types=[
            pltpu.VMEM((n_tab,), jnp.float32),
            pltpu.VMEM((n_tab,), jnp.float32),
            pltpu.VMEM((n_tab,), jnp.float32),
            pltpu.VMEM((H, W), jnp.float32),
            pltpu.VMEM((H, W), jnp.float32),
            pltpu.VMEM((H, W), jnp.float32),
            pltpu.VMEM((H, W), jnp.float32),
            pltpu.SemaphoreType.DMA,
            pltpu.SemaphoreType.DMA,
            pltpu.SemaphoreType.DMA,
            pltpu.SemaphoreType.DMA,
        ],
        compiler_params=pltpu.CompilerParams(needs_layout_passes=False),
    )
    def spline_sc(x_hbm, p0_hbm, p1_hbm, p2_hbm, out_hbm,
                  tab0, tab1, tab2, xb0, xb1, ob0, ob1,
                  sin0, sin1, sout0, sout1):
        wid = lax.axis_index("s") * 2 + lax.axis_index("c")
        pltpu.sync_copy(p0_hbm, tab0)
        pltpu.sync_copy(p1_hbm, tab1)
        pltpu.sync_copy(p2_hbm, tab2)
        page0 = wid * pages_per_w
        xbufs, obufs = (xb0, xb1), (ob0, ob1)
        sins, souts = (sin0, sin1), (sout0, sout1)
        n_pairs = pages_per_w // 2

        def page_bc(p):
            return lax.div(p, _N_CHANNELS), lax.rem(p, _N_CHANNELS)

        b0, c0 = page_bc(page0)
        pltpu.async_copy(x_hbm.at[b0, c0], xb0, sin0)
        b1, c1 = page_bc(page0 + 1)
        pltpu.async_copy(x_hbm.at[b1, c1], xb1, sin1)

        def pair_body(g, carry):
            for bi in range(2):
                page = page0 + g * 2 + bi
                pb, pc = page_bc(page)
                xbuf, obuf = xbufs[bi], obufs[bi]
                sin, sout = sins[bi], souts[bi]
                pltpu.make_async_copy(x_hbm.at[pb, pc], xbuf, sin).wait()

                @pl.when(g > 0)
                def _wait_out():
                    pltpu.make_async_copy(obuf, out_hbm.at[pb, pc], sout).wait()

                base = pc * _N_KNOTS + half - 64

                @plsc.parallel_loop(0, H, 1, unroll=4)
                def vec_body(r):
                    for cc in range(W // 16):
                        xv = xbuf[r, pl.ds(cc * 16, 16)]
                        q = xv * inv_grid
                        ti = (q + 64.0).astype(jnp.int32)
                        ti = jnp.minimum(jnp.maximum(ti, 64 - half),
                                         64 + half - 2)
                        idx = ti + base
                        p0 = plsc.load_gather(tab0, [idx])
                        p1 = plsc.load_gather(tab1, [idx])
                        p2 = plsc.load_gather(tab2, [idx])
                        obuf[r, pl.ds(cc * 16, 16)] = p0 + q * (p1 + q * p2)

                pltpu.async_copy(obuf, out_hbm.at[pb, pc], sout)

                @pl.when(g < n_pairs - 1)
                def _next_in():
                    nb, nc = page_bc(page + 2)
                    pltpu.async_copy(x_hbm.at[nb, nc], xbuf, sin)
            return carry

        lax.fori_loop(0, n_pairs, pair_body, 0)
        lb0, lc0 = page_bc(page0 + pages_per_w - 2)
        pltpu.make_async_copy(ob0, out_hbm.at[lb0, lc0], sout0).wait()
        lb1, lc1 = page_bc(page0 + pages_per_w - 1)
        pltpu.make_async_copy(ob1, out_hbm.at[lb1, lc1], sout1).wait()

    return spline_sc(x, p0_t, p1_t, p2_t)


# speedup vs baseline: 4316.4897x; 1.0004x over previous
import functools

import jax
import jax.numpy as jnp
from jax import lax
from jax.experimental import pallas as pl
from jax.experimental.pallas import tpu as pltpu
from jax.experimental.pallas import tpu_sc as plsc

_N_CHANNELS = 192
_N_KNOTS = 63
_T_RANGE = 4.0


def _grid_value():
    round_to = 1e-06
    return float(_T_RANGE) / (_N_KNOTS // 2) // round_to * round_to


def kernel(x, coefficients_vect):
    B, C, H, W = x.shape
    n_pages = B * C
    n_workers = 32
    pages_per_w = n_pages // n_workers

    grid = _grid_value()
    inv_grid = 1.0 / grid
    half = _N_KNOTS // 2
    n_tab = _N_CHANNELS * _N_KNOTS

    g0 = coefficients_vect
    g1 = jnp.concatenate([coefficients_vect[1:], jnp.zeros((1,), jnp.float32)])
    g2 = jnp.concatenate([coefficients_vect[2:], jnp.zeros((2,), jnp.float32)])
    fl = (jnp.arange(n_tab, dtype=jnp.float32) % _N_KNOTS) - float(half)
    a = 0.5 * (g0 + g1)
    b = g1 - g0
    c2 = 0.5 * (g0 + g2) - g1
    p2_t = c2
    p1_t = b - 2.0 * c2 * fl
    p0_t = a - b * fl + c2 * fl * fl

    mesh = plsc.VectorSubcoreMesh(core_axis_name="c", subcore_axis_name="s")

    @functools.partial(
        pl.kernel,
        mesh=mesh,
        out_type=jax.ShapeDtypeStruct((B, C, H, W), jnp.float32),
        scratch_types=[
            pltpu.VMEM((n_tab,), jnp.float32),
            pltpu.VMEM((n_tab,), jnp.float32),
            pltpu.VMEM((n_tab,), jnp.float32),
            pltpu.VMEM((H, W), jnp.float32),
            pltpu.VMEM((H, W), jnp.float32),
            pltpu.VMEM((H, W), jnp.float32),
            pltpu.VMEM((H, W), jnp.float32),
            pltpu.SemaphoreType.DMA,
            pltpu.SemaphoreType.DMA,
            pltpu.SemaphoreType.DMA,
            pltpu.SemaphoreType.DMA,
        ],
        compiler_params=pltpu.CompilerParams(needs_layout_passes=False),
    )
    def spline_sc(x_hbm, p0_hbm, p1_hbm, p2_hbm, out_hbm,
                  tab0, tab1, tab2, xb0, xb1, ob0, ob1,
                  sin0, sin1, sout0, sout1):
        wid = lax.axis_index("s") * 2 + lax.axis_index("c")
        pltpu.sync_copy(p0_hbm, tab0)
        pltpu.sync_copy(p1_hbm, tab1)
        pltpu.sync_copy(p2_hbm, tab2)
        xbufs, obufs = (xb0, xb1), (ob0, ob1)
        sins, souts = (sin0, sin1), (sout0, sout1)
        n_pairs = pages_per_w // 2

        wb = lax.div(wid, 2)
        c_base = lax.rem(wid, 2) * pages_per_w

        pltpu.async_copy(x_hbm.at[wb, c_base], xb0, sin0)
        pltpu.async_copy(x_hbm.at[wb, c_base + 1], xb1, sin1)

        def pair_body(g, carry):
            for bi in range(2):
                pb, pc = wb, c_base + g * 2 + bi
                xbuf, obuf = xbufs[bi], obufs[bi]
                sin, sout = sins[bi], souts[bi]
                pltpu.make_async_copy(x_hbm.at[pb, pc], xbuf, sin).wait()

                @pl.when(g > 0)
                def _wait_out():
                    pltpu.make_async_copy(obuf, out_hbm.at[pb, pc], sout).wait()

                base = pc * _N_KNOTS + half - 64

                @plsc.parallel_loop(0, H, 1, unroll=4)
                def vec_body(r):
                    for cc in range(W // 16):
                        xv = xbuf[r, pl.ds(cc * 16, 16)]
                        q = xv * inv_grid
                        ti = (q + 64.0).astype(jnp.int32)
                        ti = jnp.minimum(jnp.maximum(ti, 64 - half),
                                         64 + half - 2)
                        idx = ti + base
                        p0 = plsc.load_gather(tab0, [idx])
                        p1 = plsc.load_gather(tab1, [idx])
                        p2 = plsc.load_gather(tab2, [idx])
                        obuf[r, pl.ds(cc * 16, 16)] = p0 + q * (p1 + q * p2)

                pltpu.async_copy(obuf, out_hbm.at[pb, pc], sout)

                @pl.when(g < n_pairs - 1)
                def _next_in():
                    pltpu.async_copy(x_hbm.at[pb, pc + 2], xbuf, sin)
            return carry

        lax.fori_loop(0, n_pairs, pair_body, 0)
        lc = c_base + pages_per_w - 2
        pltpu.make_async_copy(ob0, out_hbm.at[wb, lc], sout0).wait()
        pltpu.make_async_copy(ob1, out_hbm.at[wb, lc + 1], sout1).wait()

    return spline_sc(x, p0_t, p1_t, p2_t)
